# baseline (speedup 1.0000x reference)
import functools

import jax
import jax.numpy as jnp
from jax import lax
from jax.experimental import pallas as pl
from jax.experimental.pallas import tpu as pltpu
from jax.experimental.pallas import tpu_sc as plsc

VOCAB = 1000000
D = 64
H = 50
O = 30
B = 4096
L = 50
DELTA = 1.0

NC = 2
NS = 16
NW = NC * NS

POOL_ARRAYS = 7
MLP_ARRAYS = 3
ARRAYS = POOL_ARRAYS + MLP_ARRAYS
BCHUNK = B // NW
NTASK = BCHUNK // 2

IDX_ROWS = ARRAYS * B // 2
POOL_ROWS = POOL_ARRAYS * B // 2
GATH_ROWS = MLP_ARRAYS * (B // 2) * L
GSLICE = 104


@functools.cache
def _sc_gather():
  mesh = plsc.VectorSubcoreMesh(core_axis_name="c", subcore_axis_name="s")
  return pl.kernel(
      _sc_body,
      mesh=mesh,
      compiler_params=pltpu.CompilerParams(use_tc_tiling_on_sc=False),
      out_type=[
          jax.ShapeDtypeStruct((POOL_ROWS, 2 * D), jnp.float32),
          jax.ShapeDtypeStruct((GATH_ROWS, 2 * D), jnp.float32),
      ],
      scratch_types=[
          pltpu.VMEM((NTASK, 2 * D), jnp.int32),
          pltpu.VMEM((GSLICE, D), jnp.float32),
          pltpu.VMEM((GSLICE, D), jnp.float32),
          pltpu.VMEM((NTASK, 2 * D), jnp.float32),
          pltpu.SemaphoreType.DMA,
          pltpu.SemaphoreType.DMA,
      ],
  )


def _sc_body(emb, idx2, pool_out, gath_out, idx_s, buf_a, buf_b, outc,
             sem_a, sem_b):
  wid = lax.axis_index("s") * NC + lax.axis_index("c")

  def fire(t, buf, sem):
    pltpu.make_async_copy(emb.at[idx_s.at[t, pl.ds(0, GSLICE)]], buf,
                          sem).start()

  def wait(t, buf, sem):
    pltpu.make_async_copy(emb.at[idx_s.at[t, pl.ds(0, GSLICE)]], buf,
                          sem).wait()

  def accum50(buf, start):
    def body(i, accs):
      return tuple(accs[j] + buf[start + i, pl.ds(16 * j, 16)]
                   for j in range(4))
    z = jnp.zeros((16,), jnp.float32)
    return lax.fori_loop(0, L, body, (z, z, z, z))

  def run_stage(arr, process):
    pltpu.sync_copy(idx2.at[pl.ds(NTASK * (NW * arr + wid), NTASK)], idx_s)
    fire(0, buf_a, sem_a)

    def pair(g, _):
      ta = 2 * g
      tb = ta + 1
      fire(tb, buf_b, sem_b)
      wait(ta, buf_a, sem_a)
      process(buf_a, ta)

      @pl.when(ta + 2 < BCHUNK // 2)
      def _():
        fire(ta + 2, buf_a, sem_a)

      wait(tb, buf_b, sem_b)
      process(buf_b, tb)
      return 0

    lax.fori_loop(0, NTASK // 2, pair, 0)

  for arr in range(POOL_ARRAYS):

    def pool_proc(buf, t):
      a_lo = accum50(buf, 0)
      a_hi = accum50(buf, L)
      for j in range(4):
        outc[t, pl.ds(16 * j, 16)] = a_lo[j]
        outc[t, pl.ds(D + 16 * j, 16)] = a_hi[j]

    run_stage(arr, pool_proc)
    pltpu.sync_copy(
        outc, pool_out.at[pl.ds(NTASK * (NW * arr + wid), NTASK)])

  for m in range(MLP_ARRAYS):
    prow0 = L * NTASK * (NW * m + wid)

    def mlp_proc(buf, t):
      prow = prow0 + L * t
      pltpu.sync_copy(buf.at[pl.ds(0, L)],
                      gath_out.at[pl.ds(prow, L), pl.ds(0, D)])
      pltpu.sync_copy(buf.at[pl.ds(L, L)],
                      gath_out.at[pl.ds(prow, L), pl.ds(D, D)])

    run_stage(POOL_ARRAYS + m, mlp_proc)


def _tc_body(pooled_ref, gath_ref, whw_ref, whb_ref, wpw_ref, wpb_ref,
             out_ref):
  inv = 1.0 / (L * L)
  whw = whw_ref[...]
  whb = whb_ref[...]
  wpw = wpw_ref[...]
  wpb = wpb_ref[...]

  def proj(e):
    z = lax.dot_general(e, whw, (((1,), (1,)), ((), ())),
                        preferred_element_type=jnp.float32) + whb
    h = 1.0 / (1.0 + jnp.exp(-z))
    return lax.dot_general(h, wpw, (((1,), (1,)), ((), ())),
                           preferred_element_type=jnp.float32) + wpb

  qv = pooled_ref[0]
  gq = gath_ref[0, 0].reshape(NTASK, L, 2 * D)
  gp = gath_ref[1, 0].reshape(NTASK, L, 2 * D)
  gn = gath_ref[2, 0].reshape(NTASK, L, 2 * D)

  for half in range(2):
    lo, hi = D * half, D * half + D
    qv_h = qv[:, lo:hi]
    tot = jnp.zeros((NTASK,), jnp.float32)
    for k in range(3):
      dq = jnp.sum(qv_h * pooled_ref[1 + 2 * k][:, lo:hi], axis=1)
      dn = jnp.sum(qv_h * pooled_ref[2 + 2 * k][:, lo:hi], axis=1)
      tot = tot + jnp.maximum(DELTA - inv * dq + inv * dn, 0.0)

    def rowdots(x, y):
      p = (x * y).reshape(NTASK, L, O)
      return jnp.sum(jnp.sum(p, axis=2), axis=1)

    oq = proj(gq[:, :, lo:hi].reshape(NTASK * L, D))
    op_ = proj(gp[:, :, lo:hi].reshape(NTASK * L, D))
    on_ = proj(gn[:, :, lo:hi].reshape(NTASK * L, D))
    dqp = rowdots(oq, op_)
    dqn = rowdots(oq, on_)
    out_ref[0, half, :] = tot + jnp.maximum(DELTA - dqp + dqn, 0.0)


def _tc_call(pooled3, gath4, wh_w, wh_b2, wp_w, wp_b2):
  return pl.pallas_call(
      _tc_body,
      grid=(NW,),
      in_specs=[
          pl.BlockSpec((POOL_ARRAYS, NTASK, 2 * D), lambda i: (0, i, 0)),
          pl.BlockSpec((MLP_ARRAYS, 1, NTASK * L, 2 * D),
                       lambda i: (0, i, 0, 0)),
          pl.BlockSpec((H, D), lambda i: (0, 0)),
          pl.BlockSpec((1, H), lambda i: (0, 0)),
          pl.BlockSpec((O, H), lambda i: (0, 0)),
          pl.BlockSpec((1, O), lambda i: (0, 0)),
      ],
      out_specs=pl.BlockSpec((1, 2, NTASK), lambda i: (i, 0, 0)),
      out_shape=jax.ShapeDtypeStruct((NW, 2, NTASK), jnp.float32),
  )(pooled3, gath4, wh_w, wh_b2, wp_w, wp_b2)


def kernel(q_v, q_a0, n_a0, q_a1, n_a1, q_a2, n_a2, query, pos, neg,
           emb, wh_w, wh_b, wp_w, wp_b):
  idx_all = jnp.concatenate(
      [q_v, q_a0, n_a0, q_a1, n_a1, q_a2, n_a2, query, pos, neg],
      axis=0).astype(jnp.int32)
  idx2 = jnp.pad(idx_all.reshape(IDX_ROWS, 2 * L), ((0, 0), (0, 28)))

  pooled, gath = _sc_gather()(emb, idx2)
  pooled3 = pooled.reshape(POOL_ARRAYS, B // 2, 2 * D)
  gath4 = gath.reshape(MLP_ARRAYS, NW, NTASK * L, 2 * D)

  out = _tc_call(pooled3, gath4, wh_w, wh_b.reshape(1, H),
                 wp_w, wp_b.reshape(1, O))
  return out.transpose(0, 2, 1).reshape(B)


# speedup vs baseline: 2.1981x; 2.1981x over previous
import functools

import jax
import jax.numpy as jnp
from jax import lax
from jax.experimental import pallas as pl
from jax.experimental.pallas import tpu as pltpu
from jax.experimental.pallas import tpu_sc as plsc

VOCAB = 1000000
D = 64
H = 50
O = 30
B = 4096
L = 50
DELTA = 1.0

NC = 2
NS = 16
NW = NC * NS

POOL_ARRAYS = 7
MLP_ARRAYS = 3
ARRAYS = POOL_ARRAYS + MLP_ARRAYS
BCHUNK = B // NW
NTASK = BCHUNK // 2

IDX_ROWS = ARRAYS * B // 2
POOL_ROWS = POOL_ARRAYS * B // 2
GATH_ROWS = MLP_ARRAYS * (B // 2) * L
GSLICE = 104


@functools.cache
def _sc_gather():
  mesh = plsc.VectorSubcoreMesh(core_axis_name="c", subcore_axis_name="s")
  return pl.kernel(
      _sc_body,
      mesh=mesh,
      compiler_params=pltpu.CompilerParams(use_tc_tiling_on_sc=False),
      out_type=[
          jax.ShapeDtypeStruct((POOL_ROWS, 2 * D), jnp.float32),
          jax.ShapeDtypeStruct((GATH_ROWS, 2 * D), jnp.float32),
      ],
      scratch_types=[
          pltpu.VMEM((NTASK, 2 * D), jnp.int32),
          pltpu.VMEM((GSLICE, D), jnp.float32),
          pltpu.VMEM((GSLICE, D), jnp.float32),
          pltpu.VMEM((NTASK, 2 * D), jnp.float32),
          pltpu.SemaphoreType.DMA,
          pltpu.SemaphoreType.DMA,
      ],
  )


def _sc_body(emb, idx2, pool_out, gath_out, idx_s, buf_a, buf_b, outc,
             sem_a, sem_b):
  wid = lax.axis_index("s") * NC + lax.axis_index("c")

  def fire(t, buf, sem):
    pltpu.make_async_copy(emb.at[idx_s.at[t, pl.ds(0, GSLICE)]], buf,
                          sem).start()

  def wait(t, buf, sem):
    pltpu.make_async_copy(emb.at[idx_s.at[t, pl.ds(0, GSLICE)]], buf,
                          sem).wait()

  def accum50(buf, start):
    def body(i, accs):
      return tuple(accs[j] + buf[start + i, pl.ds(16 * j, 16)]
                   for j in range(4))
    z = jnp.zeros((16,), jnp.float32)
    return lax.fori_loop(0, L, body, (z, z, z, z))

  def run_stage(arr, process):
    pltpu.sync_copy(idx2.at[pl.ds(NTASK * (NW * arr + wid), NTASK)], idx_s)
    fire(0, buf_a, sem_a)

    def pair(g, _):
      ta = 2 * g
      tb = ta + 1
      fire(tb, buf_b, sem_b)
      wait(ta, buf_a, sem_a)
      process(buf_a, ta)

      @pl.when(ta + 2 < BCHUNK // 2)
      def _():
        fire(ta + 2, buf_a, sem_a)

      wait(tb, buf_b, sem_b)
      process(buf_b, tb)
      return 0

    lax.fori_loop(0, NTASK // 2, pair, 0)

  for arr in range(POOL_ARRAYS):

    def pool_proc(buf, t):
      a_lo = accum50(buf, 0)
      a_hi = accum50(buf, L)
      for j in range(4):
        outc[t, pl.ds(16 * j, 16)] = a_lo[j]
        outc[t, pl.ds(D + 16 * j, 16)] = a_hi[j]

    run_stage(arr, pool_proc)
    pltpu.sync_copy(
        outc, pool_out.at[pl.ds(NTASK * (NW * arr + wid), NTASK)])

  for m in range(MLP_ARRAYS):
    prow0 = L * NTASK * (NW * m + wid)

    def mlp_proc(buf, t):
      prow = prow0 + L * t
      pltpu.sync_copy(buf.at[pl.ds(0, L)],
                      gath_out.at[pl.ds(prow, L), pl.ds(0, D)])
      pltpu.sync_copy(buf.at[pl.ds(L, L)],
                      gath_out.at[pl.ds(prow, L), pl.ds(D, D)])

    run_stage(POOL_ARRAYS + m, mlp_proc)


TCCH = 2
TCM = TCCH * NTASK


def _tc_body(pooled_ref, gath_ref, whw_ref, whb_ref, wpw_ref, wpb_ref,
             out_ref):
  inv = 1.0 / (L * L)
  whw = whw_ref[...]
  whb = whb_ref[...]
  wpw = wpw_ref[...]
  wpb = wpb_ref[...]

  def proj(e):
    z = lax.dot_general(e, whw, (((1,), (1,)), ((), ())),
                        preferred_element_type=jnp.float32) + whb
    h = 1.0 / (1.0 + jnp.exp(-z))
    return lax.dot_general(h, wpw, (((1,), (1,)), ((), ())),
                           preferred_element_type=jnp.float32) + wpb

  qv = pooled_ref[0]
  gq = gath_ref[0].reshape(TCM, L, 2 * D)
  gp = gath_ref[1].reshape(TCM, L, 2 * D)
  gn = gath_ref[2].reshape(TCM, L, 2 * D)

  for half in range(2):
    lo, hi = D * half, D * half + D
    qv_h = qv[:, lo:hi]
    tot = jnp.zeros((TCM,), jnp.float32)
    for k in range(3):
      dq = jnp.sum(qv_h * pooled_ref[1 + 2 * k][:, lo:hi], axis=1)
      dn = jnp.sum(qv_h * pooled_ref[2 + 2 * k][:, lo:hi], axis=1)
      tot = tot + jnp.maximum(DELTA - inv * dq + inv * dn, 0.0)

    def rowdots(x, y):
      p = (x * y).reshape(TCM, L, O)
      return jnp.sum(jnp.sum(p, axis=2), axis=1)

    oq = proj(gq[:, :, lo:hi].reshape(TCM * L, D))
    op_ = proj(gp[:, :, lo:hi].reshape(TCM * L, D))
    on_ = proj(gn[:, :, lo:hi].reshape(TCM * L, D))
    dqp = rowdots(oq, op_)
    dqn = rowdots(oq, on_)
    out_ref[0, half, :] = tot + jnp.maximum(DELTA - dqp + dqn, 0.0)


def _tc_call(pooled3, gath4, wh_w, wh_b2, wp_w, wp_b2):
  return pl.pallas_call(
      _tc_body,
      grid=(NW // TCCH,),
      in_specs=[
          pl.BlockSpec((POOL_ARRAYS, TCM, 2 * D), lambda i: (0, i, 0)),
          pl.BlockSpec((MLP_ARRAYS, TCM * L, 2 * D), lambda i: (0, i, 0)),
          pl.BlockSpec((H, D), lambda i: (0, 0)),
          pl.BlockSpec((1, H), lambda i: (0, 0)),
          pl.BlockSpec((O, H), lambda i: (0, 0)),
          pl.BlockSpec((1, O), lambda i: (0, 0)),
      ],
      out_specs=pl.BlockSpec((1, 2, TCM), lambda i: (i, 0, 0)),
      out_shape=jax.ShapeDtypeStruct((NW // TCCH, 2, TCM), jnp.float32),
  )(pooled3, gath4, wh_w, wh_b2, wp_w, wp_b2)


def kernel(q_v, q_a0, n_a0, q_a1, n_a1, q_a2, n_a2, query, pos, neg,
           emb, wh_w, wh_b, wp_w, wp_b):
  idx_all = jnp.concatenate(
      [q_v, q_a0, n_a0, q_a1, n_a1, q_a2, n_a2, query, pos, neg],
      axis=0).astype(jnp.int32)
  idx100 = idx_all.reshape(IDX_ROWS, 2 * L)
  idx2 = jnp.pad(
      jnp.concatenate([idx100, idx100[:, : GSLICE - 2 * L]], axis=1),
      ((0, 0), (0, 128 - GSLICE)))

  pooled, gath = _sc_gather()(emb, idx2)
  pooled3 = pooled.reshape(POOL_ARRAYS, B // 2, 2 * D)
  gath3 = gath.reshape(MLP_ARRAYS, NW * NTASK * L, 2 * D)

  out = _tc_call(pooled3, gath3, wh_w, wh_b.reshape(1, H),
                 wp_w, wp_b.reshape(1, O))
  return out.reshape(NW // TCCH, 2, TCCH, NTASK).transpose(0, 2, 3, 1).reshape(B)
